# baseline (device time: 79639 ns/iter reference)
import jax
import jax.numpy as jnp
from jax import lax
from jax.experimental import pallas as pl
from jax.experimental.pallas import tpu as pltpu

N_DEV = 8
R3_ROWS = ((0, 192), (192, 192), (384, 128))
W_TILES = 8
SPLIT = 4


def kernel(x, w_mat, scale_x, scale_w):
    if x.dtype != jnp.float32:
        x = x.astype(jnp.float32)
    if w_mat.dtype != jnp.float32:
        w_mat = w_mat.astype(jnp.float32)
    s = (scale_x.astype(jnp.float32) * scale_w.astype(jnp.float32)).reshape(1, 1)
    m_per, k = x.shape
    n = w_mat.shape[1]
    k_tile = k // W_TILES
    q_rows = m_per // SPLIT

    def body(x_ref, w_ref, s_ref, out_ref,
             x8_ref, w8_ref, wbuf_ref, wsems,
             comm_ref, send_sems, recv_sems, acc_ref, copy_sems):
        my = lax.axis_index("i")
        p = my & 3
        zp = my & 4
        xn = zp | (p ^ 1)
        yn = zp | (p ^ 3)
        zn = my ^ 4
        nbrs = [xn, yn, zn]
        yxn = (xn & 4) | ((xn & 3) ^ 3)
        zyn = yn ^ 4
        xzn = (zn & 4) | ((zn & 3) ^ 1)
        anti = (my & 4) ^ 4 | ((my & 3) ^ 2)

        x8_ref[:, :] = x_ref[:, :].astype(jnp.float8_e5m2)

        barrier_sem = pltpu.get_barrier_semaphore()
        for nb in nbrs:
            pl.semaphore_signal(
                barrier_sem, inc=1,
                device_id=(nb,), device_id_type=pl.DeviceIdType.MESH,
            )
        pl.semaphore_wait(barrier_sem, 3)

        def rc(src, dst, sem_i, target):
            return pltpu.make_async_remote_copy(
                src_ref=src, dst_ref=dst,
                send_sem=send_sems.at[sem_i], recv_sem=recv_sems.at[sem_i],
                device_id=(target,), device_id_type=pl.DeviceIdType.MESH,
            )

        def qslice(ref_slot, q):
            return comm_ref.at[ref_slot, pl.ds(q * q_rows, q_rows), :]


        r1 = [[rc(x8_ref.at[pl.ds(q * q_rows, q_rows), :], qslice(i, q),
                  i * SPLIT + q, nbrs[i])
               for q in range(SPLIT)] for i in range(3)]
        for q in range(SPLIT):
            for i in range(3):
                r1[i][q].start()

        wcp = []
        for t in range(2):
            cp = pltpu.make_async_copy(
                w_ref.at[pl.ds(t * k_tile, k_tile), :],
                wbuf_ref.at[t % 2], wsems.at[t % 2],
            )
            cp.start()
            wcp.append(cp)
        for t in range(W_TILES):
            wcp[t].wait()
            w8_ref[pl.ds(t * k_tile, k_tile), :] = (
                wbuf_ref[t % 2, :, :].astype(jnp.float8_e5m2))
            if t + 2 < W_TILES:
                cp = pltpu.make_async_copy(
                    w_ref.at[pl.ds((t + 2) * k_tile, k_tile), :],
                    wbuf_ref.at[t % 2], wsems.at[t % 2],
                )
                cp.start()
                wcp.append(cp)

        scale = s_ref[0, 0]

        out_copies = []

        def gemm(chunk, origin):
            i = len(out_copies)
            slot = i % 2
            if i >= 2:
                out_copies[i - 2].wait()
            acc = lax.dot_general(
                chunk, w8_ref[:, :],
                (((1,), (0,)), ((), ())),
                preferred_element_type=jnp.float32,
            )
            acc_ref[slot, :, :] = acc * scale
            cp = pltpu.make_async_copy(
                acc_ref.at[slot],
                out_ref.at[pl.ds(origin * m_per, m_per), :],
                copy_sems.at[slot],
            )
            cp.start()
            out_copies.append(cp)

        gemm(x8_ref[:, :], my)

        srcs2 = (1, 2, 0)
        gate2 = (1, 2, 0)
        r2 = [[rc(qslice(srcs2[i], q), qslice(3 + i, q),
                  12 + i * SPLIT + q, nbrs[i])
               for q in range(SPLIT)] for i in range(3)]
        for q in range(SPLIT):
            for i in range(3):
                r1[gate2[i]][q].wait_recv()
                r2[i][q].start()
        gemm(comm_ref[0, :, :], xn)
        gemm(comm_ref[1, :, :], yn)
        gemm(comm_ref[2, :, :], zn)

        srcs3 = (4, 5, 3)
        g3link = (1, 2, 0)
        r3 = []
        for i in range(3):
            st, ln = R3_ROWS[i]
            r3.append(rc(
                comm_ref.at[srcs3[i], pl.ds(st, ln), :],
                comm_ref.at[6, pl.ds(st, ln), :],
                24 + i, nbrs[i],
            ))
        pieces = sorted(
            range(3),
            key=lambda i: (R3_ROWS[i][0] + R3_ROWS[i][1] - 1) // q_rows,
        )
        r2_waited = set()
        for i in pieces:
            st, ln = R3_ROWS[i]
            for q in range(st // q_rows, (st + ln - 1) // q_rows + 1):
                if (g3link[i], q) not in r2_waited:
                    r2[g3link[i]][q].wait_recv()
                    r2_waited.add((g3link[i], q))
            r3[i].start()
        for i in range(3):
            for q in range(SPLIT):
                if (i, q) not in r2_waited:
                    r2[i][q].wait_recv()
                    r2_waited.add((i, q))
        gemm(comm_ref[3, :, :], yxn)
        gemm(comm_ref[4, :, :], zyn)
        gemm(comm_ref[5, :, :], xzn)
        for r in r3:
            r.wait_recv()
        gemm(comm_ref[6, :, :], anti)

        for group in r1 + r2:
            for r in group:
                r.wait_send()
        for r in r3:
            r.wait_send()
        out_copies[-2].wait()
        out_copies[-1].wait()

    out_shape = jax.ShapeDtypeStruct((N_DEV * m_per, n), jnp.float32)
    return pl.pallas_call(
        body,
        out_shape=out_shape,
        in_specs=[
            pl.BlockSpec(memory_space=pltpu.VMEM),
            pl.BlockSpec(memory_space=pl.ANY),
            pl.BlockSpec(memory_space=pltpu.SMEM),
        ],
        out_specs=pl.BlockSpec(memory_space=pl.ANY),
        scratch_shapes=[
            pltpu.VMEM((m_per, k), jnp.float8_e5m2),
            pltpu.VMEM((k, n), jnp.float8_e5m2),
            pltpu.VMEM((2, k // W_TILES, n), jnp.float32),
            pltpu.SemaphoreType.DMA((2,)),
            pltpu.VMEM((7, m_per, k), jnp.float8_e5m2),
            pltpu.SemaphoreType.DMA((27,)),
            pltpu.SemaphoreType.DMA((27,)),
            pltpu.VMEM((2, m_per, n), jnp.float32),
            pltpu.SemaphoreType.DMA((2,)),
        ],
        compiler_params=pltpu.CompilerParams(collective_id=0),
    )(x, w_mat, s)


# device time: 79515 ns/iter; 1.0016x vs baseline; 1.0016x over previous
import jax
import jax.numpy as jnp
from jax import lax
from jax.experimental import pallas as pl
from jax.experimental.pallas import tpu as pltpu

N_DEV = 8
R3_ROWS = ((0, 192), (192, 192), (384, 128))
W_TILES = 8
SPLIT = 4


def kernel(x, w_mat, scale_x, scale_w):
    if x.dtype != jnp.float32:
        x = x.astype(jnp.float32)
    if w_mat.dtype != jnp.float32:
        w_mat = w_mat.astype(jnp.float32)
    s = (scale_x.astype(jnp.float32) * scale_w.astype(jnp.float32)).reshape(1, 1)
    m_per, k = x.shape
    n = w_mat.shape[1]
    k_tile = k // W_TILES
    q_rows = m_per // SPLIT

    def body(x_ref, w_ref, s_ref, out_ref,
             x8_ref, w8_ref, wbuf_ref, wsems,
             comm_ref, send_sems, recv_sems, acc_ref, copy_sems):
        my = lax.axis_index("i")
        p = my & 3
        zp = my & 4
        xn = zp | (p ^ 1)
        yn = zp | (p ^ 3)
        zn = my ^ 4
        nbrs = [xn, yn, zn]
        yxn = (xn & 4) | ((xn & 3) ^ 3)
        zyn = yn ^ 4
        xzn = (zn & 4) | ((zn & 3) ^ 1)
        anti = (my & 4) ^ 4 | ((my & 3) ^ 2)

        barrier_sem = pltpu.get_barrier_semaphore()
        for nb in nbrs:
            pl.semaphore_signal(
                barrier_sem, inc=1,
                device_id=(nb,), device_id_type=pl.DeviceIdType.MESH,
            )
        pl.semaphore_wait(barrier_sem, 3)

        def rc(src, dst, sem_i, target):
            return pltpu.make_async_remote_copy(
                src_ref=src, dst_ref=dst,
                send_sem=send_sems.at[sem_i], recv_sem=recv_sems.at[sem_i],
                device_id=(target,), device_id_type=pl.DeviceIdType.MESH,
            )

        def qslice(ref_slot, q):
            return comm_ref.at[ref_slot, pl.ds(q * q_rows, q_rows), :]


        r1 = [[rc(x8_ref.at[pl.ds(q * q_rows, q_rows), :], qslice(i, q),
                  i * SPLIT + q, nbrs[i])
               for q in range(SPLIT)] for i in range(3)]
        for q in range(SPLIT):
            rows = pl.ds(q * q_rows, q_rows)
            x8_ref[rows, :] = x_ref[rows, :].astype(jnp.float8_e5m2)
            for i in range(3):
                r1[i][q].start()

        wcp = []
        for t in range(2):
            cp = pltpu.make_async_copy(
                w_ref.at[pl.ds(t * k_tile, k_tile), :],
                wbuf_ref.at[t % 2], wsems.at[t % 2],
            )
            cp.start()
            wcp.append(cp)
        for t in range(W_TILES):
            wcp[t].wait()
            w8_ref[pl.ds(t * k_tile, k_tile), :] = (
                wbuf_ref[t % 2, :, :].astype(jnp.float8_e5m2))
            if t + 2 < W_TILES:
                cp = pltpu.make_async_copy(
                    w_ref.at[pl.ds((t + 2) * k_tile, k_tile), :],
                    wbuf_ref.at[t % 2], wsems.at[t % 2],
                )
                cp.start()
                wcp.append(cp)

        scale = s_ref[0, 0]

        out_copies = []

        def gemm(chunk, origin):
            i = len(out_copies)
            slot = i % 2
            if i >= 2:
                out_copies[i - 2].wait()
            acc = lax.dot_general(
                chunk, w8_ref[:, :],
                (((1,), (0,)), ((), ())),
                preferred_element_type=jnp.float32,
            )
            acc_ref[slot, :, :] = acc * scale
            cp = pltpu.make_async_copy(
                acc_ref.at[slot],
                out_ref.at[pl.ds(origin * m_per, m_per), :],
                copy_sems.at[slot],
            )
            cp.start()
            out_copies.append(cp)

        gemm(x8_ref[:, :], my)

        srcs2 = (1, 2, 0)
        gate2 = (1, 2, 0)
        r2 = [[rc(qslice(srcs2[i], q), qslice(3 + i, q),
                  12 + i * SPLIT + q, nbrs[i])
               for q in range(SPLIT)] for i in range(3)]
        for q in range(SPLIT):
            for i in range(3):
                r1[gate2[i]][q].wait_recv()
                r2[i][q].start()
        gemm(comm_ref[0, :, :], xn)
        gemm(comm_ref[1, :, :], yn)
        gemm(comm_ref[2, :, :], zn)

        srcs3 = (4, 5, 3)
        g3link = (1, 2, 0)
        r3 = []
        for i in range(3):
            st, ln = R3_ROWS[i]
            r3.append(rc(
                comm_ref.at[srcs3[i], pl.ds(st, ln), :],
                comm_ref.at[6, pl.ds(st, ln), :],
                24 + i, nbrs[i],
            ))
        pieces = sorted(
            range(3),
            key=lambda i: (R3_ROWS[i][0] + R3_ROWS[i][1] - 1) // q_rows,
        )
        r2_waited = set()
        for i in pieces:
            st, ln = R3_ROWS[i]
            for q in range(st // q_rows, (st + ln - 1) // q_rows + 1):
                if (g3link[i], q) not in r2_waited:
                    r2[g3link[i]][q].wait_recv()
                    r2_waited.add((g3link[i], q))
            r3[i].start()
        for i in range(3):
            for q in range(SPLIT):
                if (i, q) not in r2_waited:
                    r2[i][q].wait_recv()
                    r2_waited.add((i, q))
        gemm(comm_ref[3, :, :], yxn)
        gemm(comm_ref[4, :, :], zyn)
        gemm(comm_ref[5, :, :], xzn)
        a_slot = len(out_copies) % 2
        out_copies[len(out_copies) - 2].wait()
        for i in pieces:
            st, ln = R3_ROWS[i]
            r3[i].wait_recv()
            acc = lax.dot_general(
                comm_ref[6, pl.ds(st, ln), :], w8_ref[:, :],
                (((1,), (0,)), ((), ())),
                preferred_element_type=jnp.float32,
            )
            acc_ref[a_slot, pl.ds(st, ln), :] = acc * scale
        a_cp = pltpu.make_async_copy(
            acc_ref.at[a_slot],
            out_ref.at[pl.ds(anti * m_per, m_per), :],
            copy_sems.at[a_slot],
        )
        a_cp.start()
        out_copies.append(a_cp)

        for group in r1 + r2:
            for r in group:
                r.wait_send()
        for r in r3:
            r.wait_send()
        out_copies[-2].wait()
        out_copies[-1].wait()

    out_shape = jax.ShapeDtypeStruct((N_DEV * m_per, n), jnp.float32)
    return pl.pallas_call(
        body,
        out_shape=out_shape,
        in_specs=[
            pl.BlockSpec(memory_space=pltpu.VMEM),
            pl.BlockSpec(memory_space=pl.ANY),
            pl.BlockSpec(memory_space=pltpu.SMEM),
        ],
        out_specs=pl.BlockSpec(memory_space=pl.ANY),
        scratch_shapes=[
            pltpu.VMEM((m_per, k), jnp.float8_e5m2),
            pltpu.VMEM((k, n), jnp.float8_e5m2),
            pltpu.VMEM((2, k // W_TILES, n), jnp.float32),
            pltpu.SemaphoreType.DMA((2,)),
            pltpu.VMEM((7, m_per, k), jnp.float8_e5m2),
            pltpu.SemaphoreType.DMA((27,)),
            pltpu.SemaphoreType.DMA((27,)),
            pltpu.VMEM((2, m_per, n), jnp.float32),
            pltpu.SemaphoreType.DMA((2,)),
        ],
        compiler_params=pltpu.CompilerParams(collective_id=0),
    )(x, w_mat, s)
